# baseline (speedup 1.0000x reference)
import jax
import jax.numpy as jnp
from jax import lax
from jax.experimental import pallas as pl
from jax.experimental.pallas import tpu as pltpu
from jax.experimental.pallas import tpu_sc as plsc

SEQ = 9
DM = 1024
ROWS = 2048
LANES = 16


def _sc_renorm(w_hbm, pe_hbm, row_v, sem):
    wid = lax.axis_index("s") * 2 + lax.axis_index("c")

    @pl.when(wid < SEQ)
    def _():
        pltpu.sync_copy(w_hbm.at[wid], row_v)
        acc = jnp.zeros((LANES,), jnp.float32)
        for i in range(DM // LANES):
            v = row_v[pl.ds(i * LANES, LANES)]
            acc = acc + v * v
        s0 = acc[0]
        for i in range(1, LANES):
            s0 = s0 + acc[i]
        ss = jnp.full((LANES,), s0, jnp.float32)
        y = lax.bitcast_convert_type(
            0x5F3759DF - (lax.bitcast_convert_type(ss, jnp.int32) >> 1),
            jnp.float32,
        )
        for _ in range(4):
            y = y * (1.5 - 0.5 * ss * y * y)
        norm = ss * y
        scale = jnp.where(norm > 1.0, y * (1.0 - 1e-7 * y), 1.0)
        for i in range(DM // LANES):
            sl = pl.ds(i * LANES, LANES)
            row_v[sl] = row_v[sl] * scale
        pltpu.sync_copy(row_v, pe_hbm.at[wid])


_renorm_table = pl.kernel(
    _sc_renorm,
    out_type=jax.ShapeDtypeStruct((SEQ, DM), jnp.float32),
    scratch_types=[
        pltpu.VMEM((DM,), jnp.float32),
        pltpu.SemaphoreType.DMA,
    ],
    mesh=plsc.VectorSubcoreMesh(core_axis_name="c", subcore_axis_name="s"),
)


def _tc_add(x_ref, pe_ref, o_ref):
    o_ref[...] = x_ref[...] + pe_ref[...][:, :, None, :]


def kernel(x, W):
    B = x.shape[0]
    pe = _renorm_table(W)
    pe3 = pe.reshape(SEQ, 1, DM)
    return pl.pallas_call(
        _tc_add,
        grid=(B, SEQ),
        in_specs=[
            pl.BlockSpec((1, 1, ROWS, DM), lambda b, s: (b, s, 0, 0)),
            pl.BlockSpec((1, 1, DM), lambda b, s: (s, 0, 0)),
        ],
        out_specs=pl.BlockSpec((1, 1, ROWS, DM), lambda b, s: (b, s, 0, 0)),
        out_shape=jax.ShapeDtypeStruct(x.shape, x.dtype),
        compiler_params=pltpu.CompilerParams(
            dimension_semantics=("arbitrary", "arbitrary"),
        ),
    )(x, pe3)


# speedup vs baseline: 1.2131x; 1.2131x over previous
import jax
import jax.numpy as jnp
from jax import lax
from jax.experimental import pallas as pl
from jax.experimental.pallas import tpu as pltpu

SEQ = 9
DM = 1024
SEG = 2048
BLKR = 3072
PAD = 16


def _body(x_ref, w_ref, o_ref, pe_ref):
    i = pl.program_id(0)

    @pl.when(i == 0)
    def _init():
        w = w_ref[:, 0, :]
        ss = jnp.sum(w * w, axis=-1, keepdims=True)
        norm = jnp.sqrt(ss)
        scale = jnp.where(norm > 1.0, 1.0 / (norm + 1e-7), 1.0)
        pe_ref[...] = w * scale

    base = i * BLKR
    s0 = base // SEG
    r0 = lax.rem(s0, SEQ)
    r1 = lax.rem(s0 + 1, SEQ)
    bnd = SEG - lax.rem(base, SEG)
    row0 = pe_ref[pl.ds(r0, 1), :]
    row1 = pe_ref[pl.ds(r1, 1), :]
    iota = lax.broadcasted_iota(jnp.int32, (BLKR, 1), 0)
    pe_blk = jnp.where(iota < bnd, row0, row1)
    o_ref[...] = x_ref[...] + pe_blk


def kernel(x, W):
    B = x.shape[0]
    n = B * SEQ * SEG
    xf = x.reshape(n, DM)
    Wp = jnp.zeros((PAD, 1, DM), W.dtype).at[:SEQ, 0, :].set(W)
    out = pl.pallas_call(
        _body,
        grid=(n // BLKR,),
        in_specs=[
            pl.BlockSpec((BLKR, DM), lambda i: (i, 0)),
            pl.BlockSpec((PAD, 1, DM), lambda i: (0, 0, 0)),
        ],
        out_specs=pl.BlockSpec((BLKR, DM), lambda i: (i, 0)),
        out_shape=jax.ShapeDtypeStruct((n, DM), x.dtype),
        scratch_shapes=[pltpu.VMEM((PAD, DM), jnp.float32)],
        compiler_params=pltpu.CompilerParams(
            dimension_semantics=("arbitrary",),
        ),
    )(xf, Wp)
    return out.reshape(x.shape)


# speedup vs baseline: 1.2284x; 1.0127x over previous
import jax
import jax.numpy as jnp
from jax import lax
from jax.experimental import pallas as pl
from jax.experimental.pallas import tpu as pltpu

SEQ = 9
DM = 1024
SEG = 2048
BLKR = 3840
PAD = 16


def _body(x_ref, w_ref, o_ref, pe_ref):
    i = pl.program_id(0)

    @pl.when(i == 0)
    def _init():
        w = w_ref[:, 0, :]
        ss = jnp.sum(w * w, axis=-1, keepdims=True)
        norm = jnp.sqrt(ss)
        scale = jnp.where(norm > 1.0, 1.0 / (norm + 1e-7), 1.0)
        pe_ref[...] = w * scale

    base = i * BLKR
    s0 = base // SEG
    r0 = lax.rem(s0, SEQ)
    r1 = lax.rem(s0 + 1, SEQ)
    r2 = lax.rem(s0 + 2, SEQ)
    bnd1 = SEG - lax.rem(base, SEG)
    bnd2 = bnd1 + SEG
    row0 = pe_ref[pl.ds(r0, 1), :]
    row1 = pe_ref[pl.ds(r1, 1), :]
    row2 = pe_ref[pl.ds(r2, 1), :]
    iota = lax.broadcasted_iota(jnp.int32, (BLKR, 1), 0)
    pe_blk = jnp.where(iota < bnd1, row0, jnp.where(iota < bnd2, row1, row2))
    o_ref[...] = x_ref[...] + pe_blk


def kernel(x, W):
    B = x.shape[0]
    n = B * SEQ * SEG
    xf = x.reshape(n, DM)
    Wp = jnp.zeros((PAD, 1, DM), W.dtype).at[:SEQ, 0, :].set(W)
    out = pl.pallas_call(
        _body,
        grid=((n + BLKR - 1) // BLKR,),
        in_specs=[
            pl.BlockSpec((BLKR, DM), lambda i: (i, 0)),
            pl.BlockSpec((PAD, 1, DM), lambda i: (0, 0, 0)),
        ],
        out_specs=pl.BlockSpec((BLKR, DM), lambda i: (i, 0)),
        out_shape=jax.ShapeDtypeStruct((n, DM), x.dtype),
        scratch_shapes=[pltpu.VMEM((PAD, DM), jnp.float32)],
        compiler_params=pltpu.CompilerParams(
            dimension_semantics=("arbitrary",),
            vmem_limit_bytes=66000000,
        ),
    )(xf, Wp)
    return out.reshape(x.shape)
